# baseline (speedup 1.0000x reference)
import jax
import jax.numpy as jnp
from jax.experimental import pallas as pl


def _onehot_t_body(xt_ref, o_ref):
    jb, dblk, b = o_ref.shape
    d = jax.lax.broadcasted_iota(jnp.int32, (jb, dblk, b), 1)
    x = xt_ref[...]
    o_ref[...] = (x[:, 0, None, :] == d).astype(jnp.float32)


def kernel(X_in, ones):
    B, J = X_in.shape
    depth = ones.shape[0]
    xt = X_in.T.reshape(J, 1, B)
    t = pl.pallas_call(
        _onehot_t_body,
        grid=(J,),
        in_specs=[pl.BlockSpec((1, 1, B), lambda i: (i, 0, 0))],
        out_specs=pl.BlockSpec((1, depth, B), lambda i: (i, 0, 0)),
        out_shape=jax.ShapeDtypeStruct((J, depth, B), jnp.float32),
    )(xt)
    return jnp.transpose(t, (2, 1, 0))
